# Initial kernel scaffold; baseline (speedup 1.0000x reference)
import jax
import jax.numpy as jnp
from jax.experimental import pallas as pl


def kernel(x, w1, b1, w2, b2):
    raise NotImplementedError("write your pallas kernel here")


# speedup vs baseline: 1.9260x; 1.9260x over previous
import math

import jax
import jax.numpy as jnp
from jax.experimental import pallas as pl
from jax.experimental.pallas import tpu as pltpu

_SQRT_HALF = 1.0 / math.sqrt(2.0)


def _round_up(x, m):
    return ((x + m - 1) // m) * m


def _mlp_kernel(x_ref, w1_ref, b1_ref, w2_ref, b2_ref, o_ref):
    x = x_ref[...].astype(jnp.bfloat16)
    h = jnp.dot(x, w1_ref[...], preferred_element_type=jnp.float32)
    h = h + b1_ref[...]
    h = 0.5 * h * (1.0 + jax.lax.erf(h * _SQRT_HALF))
    out = jnp.dot(h.astype(jnp.bfloat16), w2_ref[...],
                  preferred_element_type=jnp.float32)
    o_ref[...] = (out + b2_ref[...]).astype(o_ref.dtype)


def _mlp(x, w1, b1, w2, b2, *, tile_m=448):
    B, N, C_in = x.shape
    C_hid = w1.shape[1]
    C_out = w2.shape[1]
    M = B * N

    if M <= tile_m:
        tile_m = _round_up(M, 8)
    M_pad = _round_up(M, tile_m)

    x2 = x.reshape(M, C_in)
    if M_pad != M:
        x2 = jnp.pad(x2, ((0, M_pad - M), (0, 0)))

    w1b = w1.astype(jnp.bfloat16)
    w2b = w2.astype(jnp.bfloat16)
    b1_2d = b1.reshape(1, C_hid).astype(jnp.float32)
    b2_2d = b2.reshape(1, C_out).astype(jnp.float32)

    grid = (M_pad // tile_m,)

    out = pl.pallas_call(
        _mlp_kernel,
        out_shape=jax.ShapeDtypeStruct((M_pad, C_out), x.dtype),
        grid=grid,
        in_specs=[
            pl.BlockSpec((tile_m, C_in), lambda i: (i, 0)),
            pl.BlockSpec((C_in, C_hid), lambda i: (0, 0)),
            pl.BlockSpec((1, C_hid), lambda i: (0, 0)),
            pl.BlockSpec((C_hid, C_out), lambda i: (0, 0)),
            pl.BlockSpec((1, C_out), lambda i: (0, 0)),
        ],
        out_specs=pl.BlockSpec((tile_m, C_out), lambda i: (i, 0)),
        compiler_params=pltpu.CompilerParams(
            dimension_semantics=("parallel",),
            vmem_limit_bytes=100 * 1024 * 1024),
    )(x2, w1b, b1_2d, w2b, b2_2d)

    return out[:M].reshape(B, N, C_out)


def kernel(x, w1, b1, w2, b2):
    return _mlp(x, w1, b1, w2, b2)


# speedup vs baseline: 1.9263x; 1.0002x over previous
import math

import jax
import jax.numpy as jnp
from jax.experimental import pallas as pl
from jax.experimental.pallas import tpu as pltpu

_SQRT_HALF = 1.0 / math.sqrt(2.0)


def _round_up(x, m):
    return ((x + m - 1) // m) * m


def _cast_kernel(w1_ref, w2_ref, w1o_ref, w2o_ref):
    w1o_ref[...] = w1_ref[...].astype(jnp.bfloat16)
    w2o_ref[...] = w2_ref[...].astype(jnp.bfloat16)


def _cast_weights(w1, w2, *, splits=4):
    r1 = w1.shape[0] // splits
    r2 = w2.shape[0] // splits
    return pl.pallas_call(
        _cast_kernel,
        out_shape=(jax.ShapeDtypeStruct(w1.shape, jnp.bfloat16),
                   jax.ShapeDtypeStruct(w2.shape, jnp.bfloat16)),
        grid=(splits,),
        in_specs=[
            pl.BlockSpec((r1, w1.shape[1]), lambda i: (i, 0)),
            pl.BlockSpec((r2, w2.shape[1]), lambda i: (i, 0)),
        ],
        out_specs=(
            pl.BlockSpec((r1, w1.shape[1]), lambda i: (i, 0)),
            pl.BlockSpec((r2, w2.shape[1]), lambda i: (i, 0)),
        ),
        compiler_params=pltpu.CompilerParams(
            dimension_semantics=("parallel",)),
    )(w1, w2)


def _mlp_kernel(x_ref, w1_ref, b1_ref, w2_ref, b2_ref, o_ref):
    x = x_ref[...].astype(jnp.bfloat16)
    h = jnp.dot(x, w1_ref[...], preferred_element_type=jnp.float32)
    h = h + b1_ref[...]
    h = 0.5 * h * (1.0 + jax.lax.erf(h * _SQRT_HALF))
    out = jnp.dot(h.astype(jnp.bfloat16), w2_ref[...],
                  preferred_element_type=jnp.float32)
    o_ref[...] = (out + b2_ref[...]).astype(o_ref.dtype)


def _mlp(x, w1, b1, w2, b2, *, tile_m=448):
    B, N, C_in = x.shape
    C_hid = w1.shape[1]
    C_out = w2.shape[1]
    M = B * N

    if M <= tile_m:
        tile_m = _round_up(M, 8)
    M_pad = _round_up(M, tile_m)

    x2 = x.reshape(M, C_in)
    if M_pad != M:
        x2 = jnp.pad(x2, ((0, M_pad - M), (0, 0)))

    w1b, w2b = _cast_weights(w1, w2)
    b1_2d = b1.reshape(1, C_hid).astype(jnp.float32)
    b2_2d = b2.reshape(1, C_out).astype(jnp.float32)

    grid = (M_pad // tile_m,)

    out = pl.pallas_call(
        _mlp_kernel,
        out_shape=jax.ShapeDtypeStruct((M_pad, C_out), x.dtype),
        grid=grid,
        in_specs=[
            pl.BlockSpec((tile_m, C_in), lambda i: (i, 0)),
            pl.BlockSpec((C_in, C_hid), lambda i: (0, 0)),
            pl.BlockSpec((1, C_hid), lambda i: (0, 0)),
            pl.BlockSpec((C_hid, C_out), lambda i: (0, 0)),
            pl.BlockSpec((1, C_out), lambda i: (0, 0)),
        ],
        out_specs=pl.BlockSpec((tile_m, C_out), lambda i: (i, 0)),
        compiler_params=pltpu.CompilerParams(
            dimension_semantics=("parallel",),
            vmem_limit_bytes=100 * 1024 * 1024),
    )(x2, w1b, b1_2d, w2b, b2_2d)

    return out[:M].reshape(B, N, C_out)


def kernel(x, w1, b1, w2, b2):
    return _mlp(x, w1, b1, w2, b2)


# speedup vs baseline: 2.6984x; 1.4008x over previous
import math

import jax
import jax.numpy as jnp
from jax.experimental import pallas as pl
from jax.experimental.pallas import tpu as pltpu

_SQRT_HALF = 1.0 / math.sqrt(2.0)


def _cast_kernel(w1_ref, w2_ref, w1o_ref, w2o_ref):
    w1o_ref[...] = w1_ref[...].astype(jnp.bfloat16)
    w2o_ref[...] = w2_ref[...].astype(jnp.bfloat16)


def _cast_weights(w1, w2, *, splits=4):
    r1 = w1.shape[0] // splits
    r2 = w2.shape[0] // splits
    return pl.pallas_call(
        _cast_kernel,
        out_shape=(jax.ShapeDtypeStruct(w1.shape, jnp.bfloat16),
                   jax.ShapeDtypeStruct(w2.shape, jnp.bfloat16)),
        grid=(splits,),
        in_specs=[
            pl.BlockSpec((r1, w1.shape[1]), lambda i: (i, 0)),
            pl.BlockSpec((r2, w2.shape[1]), lambda i: (i, 0)),
        ],
        out_specs=(
            pl.BlockSpec((r1, w1.shape[1]), lambda i: (i, 0)),
            pl.BlockSpec((r2, w2.shape[1]), lambda i: (i, 0)),
        ),
        compiler_params=pltpu.CompilerParams(
            dimension_semantics=("parallel",)),
    )(w1, w2)


def _mlp_kernel(x_ref, w1_ref, b1_ref, w2_ref, b2_ref, o_ref):
    nb = x_ref.shape[0]
    for b in range(nb):
        xb = x_ref[b].astype(jnp.bfloat16)
        h = jnp.dot(xb, w1_ref[...], preferred_element_type=jnp.float32)
        h = h + b1_ref[...]
        h = 0.5 * h * (1.0 + jax.lax.erf(h * _SQRT_HALF))
        out = jnp.dot(h.astype(jnp.bfloat16), w2_ref[...],
                      preferred_element_type=jnp.float32)
        o_ref[b] = (out + b2_ref[...]).astype(o_ref.dtype)


def _mlp(x, w1, b1, w2, b2, *, tile_b=4):
    B, N, C_in = x.shape
    C_hid = w1.shape[1]
    C_out = w2.shape[1]

    w1b, w2b = _cast_weights(w1, w2)
    b1_2d = b1.reshape(1, C_hid).astype(jnp.float32)
    b2_2d = b2.reshape(1, C_out).astype(jnp.float32)

    grid = (B // tile_b,)

    return pl.pallas_call(
        _mlp_kernel,
        out_shape=jax.ShapeDtypeStruct((B, N, C_out), x.dtype),
        grid=grid,
        in_specs=[
            pl.BlockSpec((tile_b, N, C_in), lambda i: (i, 0, 0)),
            pl.BlockSpec((C_in, C_hid), lambda i: (0, 0)),
            pl.BlockSpec((1, C_hid), lambda i: (0, 0)),
            pl.BlockSpec((C_hid, C_out), lambda i: (0, 0)),
            pl.BlockSpec((1, C_out), lambda i: (0, 0)),
        ],
        out_specs=pl.BlockSpec((tile_b, N, C_out), lambda i: (i, 0, 0)),
        compiler_params=pltpu.CompilerParams(
            dimension_semantics=("parallel",),
            vmem_limit_bytes=100 * 1024 * 1024),
    )(x, w1b, b1_2d, w2b, b2_2d)


def kernel(x, w1, b1, w2, b2):
    return _mlp(x, w1, b1, w2, b2)


# speedup vs baseline: 2.7762x; 1.0288x over previous
import math

import jax
import jax.numpy as jnp
from jax.experimental import pallas as pl
from jax.experimental.pallas import tpu as pltpu

_SQRT_HALF = 1.0 / math.sqrt(2.0)


def _mlp_kernel(x_ref, w1_ref, b1_ref, w2_ref, b2_ref, o_ref,
                w1b_ref, w2b_ref):
    @pl.when(pl.program_id(0) == 0)
    def _():
        w1b_ref[...] = w1_ref[...].astype(jnp.bfloat16)
        w2b_ref[...] = w2_ref[...].astype(jnp.bfloat16)

    nb = x_ref.shape[0]
    for b in range(nb):
        xb = x_ref[b].astype(jnp.bfloat16)
        h = jnp.dot(xb, w1b_ref[...], preferred_element_type=jnp.float32)
        h = h + b1_ref[...]
        h = 0.5 * h * (1.0 + jax.lax.erf(h * _SQRT_HALF))
        out = jnp.dot(h.astype(jnp.bfloat16), w2b_ref[...],
                      preferred_element_type=jnp.float32)
        o_ref[b] = (out + b2_ref[...]).astype(o_ref.dtype)


def _mlp(x, w1, b1, w2, b2, *, tile_b=4):
    B, N, C_in = x.shape
    C_hid = w1.shape[1]
    C_out = w2.shape[1]

    b1_2d = b1.reshape(1, C_hid).astype(jnp.float32)
    b2_2d = b2.reshape(1, C_out).astype(jnp.float32)

    grid = (B // tile_b,)

    return pl.pallas_call(
        _mlp_kernel,
        out_shape=jax.ShapeDtypeStruct((B, N, C_out), x.dtype),
        grid=grid,
        in_specs=[
            pl.BlockSpec((tile_b, N, C_in), lambda i: (i, 0, 0)),
            pl.BlockSpec((C_in, C_hid), lambda i: (0, 0)),
            pl.BlockSpec((1, C_hid), lambda i: (0, 0)),
            pl.BlockSpec((C_hid, C_out), lambda i: (0, 0)),
            pl.BlockSpec((1, C_out), lambda i: (0, 0)),
        ],
        out_specs=pl.BlockSpec((tile_b, N, C_out), lambda i: (i, 0, 0)),
        scratch_shapes=[
            pltpu.VMEM((C_in, C_hid), jnp.bfloat16),
            pltpu.VMEM((C_hid, C_out), jnp.bfloat16),
        ],
        compiler_params=pltpu.CompilerParams(
            dimension_semantics=("arbitrary",),
            vmem_limit_bytes=56 * 1024 * 1024),
    )(x, w1, b1_2d, w2, b2_2d)


def kernel(x, w1, b1, w2, b2):
    return _mlp(x, w1, b1, w2, b2)


# speedup vs baseline: 2.8888x; 1.0406x over previous
import math

import jax
import jax.numpy as jnp
from jax.experimental import pallas as pl
from jax.experimental.pallas import tpu as pltpu

_SQRT_HALF = 1.0 / math.sqrt(2.0)


def _mlp_kernel(x_ref, w1_ref, b1_ref, w2_ref, b2_ref, o_ref,
                w1b_ref, w2b_ref):
    @pl.when(pl.program_id(0) == 0)
    def _():
        w1b_ref[...] = w1_ref[...].astype(jnp.bfloat16)
        w2b_ref[...] = w2_ref[...].astype(jnp.bfloat16)

    nb, n, _ = x_ref.shape
    xcat = jnp.concatenate([x_ref[b] for b in range(nb)],
                           axis=0).astype(jnp.bfloat16)
    h = jnp.dot(xcat, w1b_ref[...], preferred_element_type=jnp.float32)
    h = h + b1_ref[...]
    h = 0.5 * h * (1.0 + jax.lax.erf(h * _SQRT_HALF))
    out = jnp.dot(h.astype(jnp.bfloat16), w2b_ref[...],
                  preferred_element_type=jnp.float32)
    out = out + b2_ref[...]
    for b in range(nb):
        o_ref[b] = out[b * n:(b + 1) * n].astype(o_ref.dtype)


def _mlp(x, w1, b1, w2, b2, *, tile_b=4):
    B, N, C_in = x.shape
    C_hid = w1.shape[1]
    C_out = w2.shape[1]

    b1_2d = b1.reshape(1, C_hid).astype(jnp.float32)
    b2_2d = b2.reshape(1, C_out).astype(jnp.float32)

    grid = (B // tile_b,)

    return pl.pallas_call(
        _mlp_kernel,
        out_shape=jax.ShapeDtypeStruct((B, N, C_out), x.dtype),
        grid=grid,
        in_specs=[
            pl.BlockSpec((tile_b, N, C_in), lambda i: (i, 0, 0)),
            pl.BlockSpec((C_in, C_hid), lambda i: (0, 0)),
            pl.BlockSpec((1, C_hid), lambda i: (0, 0)),
            pl.BlockSpec((C_hid, C_out), lambda i: (0, 0)),
            pl.BlockSpec((1, C_out), lambda i: (0, 0)),
        ],
        out_specs=pl.BlockSpec((tile_b, N, C_out), lambda i: (i, 0, 0)),
        scratch_shapes=[
            pltpu.VMEM((C_in, C_hid), jnp.bfloat16),
            pltpu.VMEM((C_hid, C_out), jnp.bfloat16),
        ],
        compiler_params=pltpu.CompilerParams(
            dimension_semantics=("arbitrary",),
            vmem_limit_bytes=56 * 1024 * 1024),
    )(x, w1, b1_2d, w2, b2_2d)


def kernel(x, w1, b1, w2, b2):
    return _mlp(x, w1, b1, w2, b2)
